# baseline (speedup 1.0000x reference)
import functools

import jax
import jax.numpy as jnp
from jax import lax
from jax.experimental import pallas as pl
from jax.experimental.pallas import tpu as pltpu
from jax.experimental.pallas import tpu_sc as plsc

_NUM_CB = 4
_K = 1024
_D = 64
_N = 16384
_TB = 2048
_BETA = 0.25
_NBLK = _N // _TB


def _argmin_idx(residual, ncb, w2row):
    rw = lax.dot_general(residual, ncb, (((1,), (1,)), ((), ())),
                         preferred_element_type=jnp.float32)
    rsq = jnp.sum(residual * residual, axis=1, keepdims=True)
    d = (rsq + w2row) + rw
    dmin = jnp.min(d, axis=1, keepdims=True)
    iota = lax.broadcasted_iota(jnp.int32, (_TB, _K), 1)
    return jnp.min(jnp.where(d <= dmin, iota, _K), axis=1, keepdims=True)


def _rot_dist_block(xb_ref, rot_ref, ncb_ref, w2_ref, r_ref, idx_ref):
    xt = jnp.concatenate(
        [jnp.transpose(xb_ref[b], (1, 0)) for b in range(_TB // 1024)],
        axis=0)
    r = lax.dot_general(xt, rot_ref[...], (((1,), (0,)), ((), ())),
                        preferred_element_type=jnp.float32)
    r_ref[...] = r
    idx_ref[...] = _argmin_idx(r, ncb_ref[...], w2_ref[...])


def _dist_block(rin_ref, q_ref, ncb_ref, w2_ref, r_ref, idx_ref):
    residual = rin_ref[...] - q_ref[...]
    r_ref[...] = residual
    idx_ref[...] = _argmin_idx(residual, ncb_ref[...], w2_ref[...])


def _final_block(xraw_ref, q0_ref, q1_ref, q2_ref, q3_ref,
                 i0_ref, i1_ref, i2_ref, i3_ref, out_ref, part_ref):
    qsum = ((q0_ref[...] + q1_ref[...]) + q2_ref[...]) + q3_ref[...]
    xraw = xraw_ref[...]
    out_ref[...] = xraw + (qsum - xraw)

    idxs = [i0_ref[...], i1_ref[...], i2_ref[...], i3_ref[...]]
    ent = jnp.zeros((_TB, 1), jnp.float32)
    for j in range(_NUM_CB):
        m = jnp.zeros((_TB, 1), jnp.float32)
        for k in range(_NUM_CB):
            m = m + (idxs[j] == idxs[k]).astype(jnp.float32)
        ent = ent - 0.25 * jnp.log(m * 0.25 + 1e-10)
    ent_sum = jnp.sum(ent)

    diff = qsum - xraw
    sq_sum = jnp.sum(diff * diff)

    lane = lax.broadcasted_iota(jnp.int32, (1, 128), 1)
    row = jnp.where(lane == 0, sq_sum, jnp.where(lane == 1, ent_sum, 0.0))
    part_ref[...] = row.reshape(1, 1, 128)


def _tc_rot_dist(xb, rot, ncb0, w20):
    return pl.pallas_call(
        _rot_dist_block,
        grid=(_NBLK,),
        in_specs=[
            pl.BlockSpec((_TB // 1024, _D, 1024), lambda i: (i, 0, 0)),
            pl.BlockSpec((_D, _D), lambda i: (0, 0)),
            pl.BlockSpec((_K, _D), lambda i: (0, 0)),
            pl.BlockSpec((1, _K), lambda i: (0, 0)),
        ],
        out_specs=[
            pl.BlockSpec((_TB, _D), lambda i: (i, 0)),
            pl.BlockSpec((_TB, 1), lambda i: (i, 0)),
        ],
        out_shape=[
            jax.ShapeDtypeStruct((_N, _D), jnp.float32),
            jax.ShapeDtypeStruct((_N, 1), jnp.int32),
        ],
    )(xb, rot, ncb0, w20)


def _tc_dist(rin, q, ncbi, w2i):
    return pl.pallas_call(
        _dist_block,
        grid=(_NBLK,),
        in_specs=[
            pl.BlockSpec((_TB, _D), lambda i: (i, 0)),
            pl.BlockSpec((_TB, _D), lambda i: (i, 0)),
            pl.BlockSpec((_K, _D), lambda i: (0, 0)),
            pl.BlockSpec((1, _K), lambda i: (0, 0)),
        ],
        out_specs=[
            pl.BlockSpec((_TB, _D), lambda i: (i, 0)),
            pl.BlockSpec((_TB, 1), lambda i: (i, 0)),
        ],
        out_shape=[
            jax.ShapeDtypeStruct((_N, _D), jnp.float32),
            jax.ShapeDtypeStruct((_N, 1), jnp.int32),
        ],
    )(rin, q, ncbi, w2i)


def _tc_final(xraw, qs, idxs):
    return pl.pallas_call(
        _final_block,
        grid=(_NBLK,),
        in_specs=(
            [pl.BlockSpec((_TB, _D), lambda i: (i, 0))] * 5
            + [pl.BlockSpec((_TB, 1), lambda i: (i, 0))] * 4
        ),
        out_specs=[
            pl.BlockSpec((_TB, _D), lambda i: (i, 0)),
            pl.BlockSpec((1, 1, 128), lambda i: (i, 0, 0)),
        ],
        out_shape=[
            jax.ShapeDtypeStruct((_N, _D), jnp.float32),
            jax.ShapeDtypeStruct((_NBLK, 1, 128), jnp.float32),
        ],
    )(xraw, *qs, *idxs)


def _make_sc_gather():
    info = plsc.get_sparse_core_info()
    nw = info.num_cores * info.num_subcores
    b_per_w = _N // nw
    chunks = b_per_w // 128
    mesh = plsc.VectorSubcoreMesh(core_axis_name="c", subcore_axis_name="s")

    @functools.partial(
        pl.kernel, mesh=mesh,
        compiler_params=pltpu.CompilerParams(use_tc_tiling_on_sc=False),
        out_type=jax.ShapeDtypeStruct((_N, _D), jnp.float32),
        scratch_types=[
            pltpu.VMEM((chunks, 128), jnp.int32),
            pltpu.VMEM((b_per_w, _D), jnp.float32),
            pltpu.SemaphoreType.DMA,
        ],
    )
    def g(cb_hbm, idx_hbm, out_hbm, idx_v, rows_v, sem):
        wid = lax.axis_index("s") * info.num_cores + lax.axis_index("c")
        pltpu.sync_copy(idx_hbm.at[pl.ds(wid * chunks, chunks)], idx_v)
        for j in range(chunks):
            pltpu.async_copy(cb_hbm.at[idx_v.at[j]],
                             rows_v.at[pl.ds(j * 128, 128)], sem).wait()
        pltpu.sync_copy(rows_v, out_hbm.at[pl.ds(wid * b_per_w, b_per_w)])

    return g


def kernel(x, rotation_matrix, codebooks):
    B, C, H, W = x.shape
    xb = x.reshape(B, C, H * W)
    xraw = x.reshape(_N, _D)
    w2 = jnp.sum(codebooks * codebooks, axis=-1)[:, None, :]
    ncb = codebooks * jnp.float32(-2.0)
    sc_gather = _make_sc_gather()

    qs, idxs = [], []
    r, idx0 = _tc_rot_dist(xb, rotation_matrix, ncb[0], w2[0])
    idx = idx0
    for i in range(_NUM_CB):
        q = sc_gather(codebooks[i], idx.reshape(128, 128))
        qs.append(q)
        idxs.append(idx)
        if i + 1 < _NUM_CB:
            r, idx = _tc_dist(r, q, ncb[i + 1], w2[i + 1])

    qout, part = _tc_final(xraw, qs, idxs)
    sq_sum = jnp.sum(part[:, 0, 0])
    ent_sum = jnp.sum(part[:, 0, 1])
    loss = (1.0 + _BETA) * sq_sum / jnp.float32(x.size)
    perplexity = jnp.exp(ent_sum)
    quant_out = qout.reshape(x.shape)
    return loss, quant_out, perplexity


# speedup vs baseline: 1.4434x; 1.4434x over previous
import jax
import jax.numpy as jnp
from jax import lax
from jax.experimental import pallas as pl
from jax.experimental.pallas import tpu as pltpu

_NUM_CB = 4
_K = 1024
_D = 64
_TB = 2048
_BETA = 0.25


def _vq_block(xb_ref, xraw_ref, rot_ref, cb_ref, ncb_ref, w2_ref,
              out_ref, part_ref):
    xt = jnp.concatenate(
        [jnp.transpose(xb_ref[b], (1, 0)) for b in range(_TB // 1024)],
        axis=0)
    xr = lax.dot_general(xt, rot_ref[...], (((1,), (0,)), ((), ())),
                         preferred_element_type=jnp.float32)
    residual = xr
    qsum = jnp.zeros_like(xr)
    iota = lax.broadcasted_iota(jnp.int32, (_TB, _K), 1).astype(jnp.float32)
    idxs = []
    for i in range(_NUM_CB):
        cb = cb_ref[i]
        w2row = w2_ref[i]
        rw = lax.dot_general(residual, ncb_ref[i], (((1,), (1,)), ((), ())),
                             preferred_element_type=jnp.float32)
        rsq = jnp.sum(residual * residual, axis=1, keepdims=True)
        d = (rsq + w2row) + rw
        dmin = jnp.min(d, axis=1, keepdims=True)
        idx = jnp.min(jnp.where(d <= dmin, iota, jnp.float32(_K)),
                      axis=1, keepdims=True)
        oh = (iota == idx).astype(jnp.float32)
        qi = lax.dot_general(oh, cb, (((1,), (0,)), ((), ())),
                             preferred_element_type=jnp.float32)
        residual = residual - qi
        qsum = qsum + qi
        idxs.append(idx)

    xraw = xraw_ref[...]
    out_ref[...] = xraw + (qsum - xraw)

    prod = jnp.ones((_TB, 1), jnp.float32)
    for j in range(_NUM_CB):
        m = jnp.zeros((_TB, 1), jnp.float32)
        for k in range(_NUM_CB):
            m = m + (idxs[j] == idxs[k]).astype(jnp.float32)
        prod = prod * (m * 0.25 + 1e-10)
    ent_sum = -0.25 * jnp.sum(jnp.log(prod))

    diff = qsum - xraw
    sq_sum = jnp.sum(diff * diff)

    lane = lax.broadcasted_iota(jnp.int32, (1, 128), 1)
    row = jnp.where(lane == 0, sq_sum, jnp.where(lane == 1, ent_sum, 0.0))
    part_ref[...] = row.reshape(1, 1, 128)


def kernel(x, rotation_matrix, codebooks):
    B, C, H, W = x.shape
    N = B * H * W
    xb = x.reshape(B, C, H * W)
    xraw = x.reshape(N, _D)
    w2 = jnp.sum(codebooks * codebooks, axis=-1)[:, None, :]
    ncb = codebooks * jnp.float32(-2.0)
    nblk = N // _TB
    q, part = pl.pallas_call(
        _vq_block,
        grid=(nblk,),
        in_specs=[
            pl.BlockSpec((_TB // 1024, _D, 1024), lambda i: (i, 0, 0)),
            pl.BlockSpec((_TB, _D), lambda i: (i, 0)),
            pl.BlockSpec((_D, _D), lambda i: (0, 0)),
            pl.BlockSpec((_NUM_CB, _K, _D), lambda i: (0, 0, 0)),
            pl.BlockSpec((_NUM_CB, _K, _D), lambda i: (0, 0, 0)),
            pl.BlockSpec((_NUM_CB, 1, _K), lambda i: (0, 0, 0)),
        ],
        out_specs=[
            pl.BlockSpec((_TB, _D), lambda i: (i, 0)),
            pl.BlockSpec((1, 1, 128), lambda i: (i, 0, 0)),
        ],
        out_shape=[
            jax.ShapeDtypeStruct((N, _D), jnp.float32),
            jax.ShapeDtypeStruct((nblk, 1, 128), jnp.float32),
        ],
        compiler_params=pltpu.CompilerParams(
            dimension_semantics=("parallel",)),
    )(xb, xraw, rotation_matrix, codebooks, ncb, w2)
    sq_sum = jnp.sum(part[:, 0, 0])
    ent_sum = jnp.sum(part[:, 0, 1])
    loss = (1.0 + _BETA) * sq_sum / jnp.float32(x.size)
    perplexity = jnp.exp(ent_sum)
    quant_out = q.reshape(x.shape)
    return loss, quant_out, perplexity
